# baseline (speedup 1.0000x reference)
import jax
import jax.numpy as jnp
from jax import lax
from jax.experimental import pallas as pl
from jax.experimental.pallas import tpu as pltpu

_N_LAYERS = 20
_LPS = 4
_N_CHAIN = _N_LAYERS // _LPS
_TOKENS = 4096
_F = 1024
_AF = _F + 8
_APPLY_TILE = 512
_N_APPLY = _TOKENS // _APPLY_TILE

_NT = (((1,), (1,)), ((), ()))
_NN = (((1,), (0,)), ((), ()))


def _bf(v):
    return v.astype(jnp.bfloat16)


def _fold(a, w, brow):
    t = lax.dot_general(_bf(a), _bf(w), _NT,
                        preferred_element_type=jnp.float32)
    return jnp.concatenate([t[0:_F, :], t[_F:, :] + brow], axis=0)


def _body(x_ref, w0_ref, w1_ref, w2_ref, w3_ref,
          b0_ref, b1_ref, b2_ref, b3_ref, out_ref, a_scr):
    i = pl.program_id(0)

    @pl.when(i == 0)
    def _init():
        a = jnp.concatenate(
            [w0_ref[0].T, jnp.broadcast_to(b0_ref[0], (_AF - _F, _F))],
            axis=0)
        a = _fold(a, w1_ref[0], b1_ref[0])
        a = _fold(a, w2_ref[0], b2_ref[0])
        a_scr[...] = _fold(a, w3_ref[0], b3_ref[0])

    @pl.when((i > 0) & (i < _N_CHAIN))
    def _chain():
        a = _fold(a_scr[...], w0_ref[0], b0_ref[0])
        a = _fold(a, w1_ref[0], b1_ref[0])
        a = _fold(a, w2_ref[0], b2_ref[0])
        a_scr[...] = _fold(a, w3_ref[0], b3_ref[0])

    @pl.when(i >= _N_CHAIN)
    def _apply():
        out_ref[...] = lax.dot_general(
            _bf(x_ref[...]), _bf(a_scr[0:_F, :]), _NN,
            preferred_element_type=jnp.float32) + a_scr[_F:_F + 1, :]


def kernel(x, W, b):
    b3 = b.reshape(_N_LAYERS, 1, _F)

    def _wspec(j):
        return pl.BlockSpec(
            (1, _F, _F),
            lambda i, j=j: (_LPS * jnp.minimum(i, _N_CHAIN - 1) + j, 0, 0))

    def _bspec(j):
        return pl.BlockSpec(
            (1, 1, _F),
            lambda i, j=j: (_LPS * jnp.minimum(i, _N_CHAIN - 1) + j, 0, 0))

    return pl.pallas_call(
        _body,
        grid=(_N_CHAIN + _N_APPLY,),
        in_specs=[
            pl.BlockSpec((_APPLY_TILE, _F),
                         lambda i: (jnp.maximum(i - _N_CHAIN, 0), 0)),
            _wspec(0), _wspec(1), _wspec(2), _wspec(3),
            _bspec(0), _bspec(1), _bspec(2), _bspec(3),
        ],
        out_specs=pl.BlockSpec((_APPLY_TILE, _F),
                               lambda i: (jnp.maximum(i - _N_CHAIN, 0), 0)),
        out_shape=jax.ShapeDtypeStruct((_TOKENS, _F), jnp.float32),
        scratch_shapes=[
            pltpu.VMEM((_AF, _F), jnp.float32),
        ],
    )(x, W, W, W, W, b3, b3, b3, b3)


# speedup vs baseline: 1.0281x; 1.0281x over previous
import jax
import jax.numpy as jnp
from jax import lax
from jax.experimental import pallas as pl
from jax.experimental.pallas import tpu as pltpu

_N_LAYERS = 20
_LPS = 4
_N_CHAIN = _N_LAYERS // _LPS
_TOKENS = 4096
_F = 1024
_AF = _F + 8
_APPLY_TILE = 1024
_N_APPLY = _TOKENS // _APPLY_TILE

_NT = (((1,), (1,)), ((), ()))
_NN = (((1,), (0,)), ((), ()))


def _bf(v):
    return v.astype(jnp.bfloat16)


def _fold(a, w, brow):
    t = lax.dot_general(_bf(a), _bf(w), _NT,
                        preferred_element_type=jnp.float32)
    return jnp.concatenate([t[0:_F, :], t[_F:, :] + brow], axis=0)


def _body(x_ref, w0_ref, w1_ref, w2_ref, w3_ref,
          b0_ref, b1_ref, b2_ref, b3_ref, out_ref, a_scr):
    i = pl.program_id(0)

    @pl.when(i == 0)
    def _init():
        a = jnp.concatenate(
            [w0_ref[0].T, jnp.broadcast_to(b0_ref[0], (_AF - _F, _F))],
            axis=0)
        a = _fold(a, w1_ref[0], b1_ref[0])
        a = _fold(a, w2_ref[0], b2_ref[0])
        a_scr[...] = _fold(a, w3_ref[0], b3_ref[0])

    @pl.when((i > 0) & (i < _N_CHAIN))
    def _chain():
        a = _fold(a_scr[...], w0_ref[0], b0_ref[0])
        a = _fold(a, w1_ref[0], b1_ref[0])
        a = _fold(a, w2_ref[0], b2_ref[0])
        a_scr[...] = _fold(a, w3_ref[0], b3_ref[0])

    @pl.when(i >= _N_CHAIN)
    def _apply():
        out_ref[...] = lax.dot_general(
            _bf(x_ref[...]), _bf(a_scr[0:_F, :]), _NN,
            preferred_element_type=jnp.float32) + a_scr[_F:_F + 1, :]


def kernel(x, W, b):
    b3 = b.reshape(_N_LAYERS, 1, _F)

    def _wspec(j):
        return pl.BlockSpec(
            (1, _F, _F),
            lambda i, j=j: (_LPS * jnp.minimum(i, _N_CHAIN - 1) + j, 0, 0))

    def _bspec(j):
        return pl.BlockSpec(
            (1, 1, _F),
            lambda i, j=j: (_LPS * jnp.minimum(i, _N_CHAIN - 1) + j, 0, 0))

    return pl.pallas_call(
        _body,
        grid=(_N_CHAIN + _N_APPLY,),
        in_specs=[
            pl.BlockSpec((_APPLY_TILE, _F),
                         lambda i: (jnp.maximum(i - _N_CHAIN, 0), 0)),
            _wspec(0), _wspec(1), _wspec(2), _wspec(3),
            _bspec(0), _bspec(1), _bspec(2), _bspec(3),
        ],
        out_specs=pl.BlockSpec((_APPLY_TILE, _F),
                               lambda i: (jnp.maximum(i - _N_CHAIN, 0), 0)),
        out_shape=jax.ShapeDtypeStruct((_TOKENS, _F), jnp.float32),
        scratch_shapes=[
            pltpu.VMEM((_AF, _F), jnp.float32),
        ],
    )(x, W, W, W, W, b3, b3, b3, b3)
